# baseline (speedup 1.0000x reference)
import functools

import jax
import jax.numpy as jnp
from jax import lax
from jax.experimental import pallas as pl
from jax.experimental.pallas import tpu as pltpu
from jax.experimental.pallas import tpu_sc as plsc

NUM_TABLES = 26
VOCAB = 100000
EMBED = 32
BATCH = 4096
OUT_D = NUM_TABLES * EMBED
L = 16

FPSC = NUM_TABLES // 2
CHUNK = 3200
MAXBASE = 96768
HICAP = MAXBASE + CHUNK
TAIL = VOCAB - HICAP
ROWS_SC = FPSC * (BATCH // 4)
PASSES = tuple((i, 1) for i in range(FPSC))
PROWS = BATCH // 4
TRASH = PROWS
ZB_ROW = PROWS + 8
CAP = BATCH

_info = plsc.get_sparse_core_info()
_NS = _info.num_subcores

_mesh = plsc.VectorSubcoreMesh(core_axis_name="c", subcore_axis_name="s")


def _splat(x):
    return jnp.full((L,), x, jnp.int32)


@functools.partial(
    pl.kernel,
    mesh=_mesh,
    out_type=jax.ShapeDtypeStruct((2 * ROWS_SC, 4 * EMBED), jnp.float32),
    scratch_types=[
        pltpu.VMEM((L, CHUNK), jnp.float32),
        pltpu.VMEM((L, CHUNK), jnp.float32),
        pltpu.VMEM((TAIL // 4, 4 * EMBED), jnp.float32),
        pltpu.VMEM((32, 128), jnp.int32),
        pltpu.VMEM((CAP,), jnp.int32),
        pltpu.VMEM((CAP,), jnp.int32),
        pltpu.VMEM((L, 4 * EMBED), jnp.float32),
        pltpu.VMEM((L, 4 * EMBED), jnp.float32),
        pltpu.VMEM_SHARED((PROWS + 8 + L, 4 * EMBED), jnp.float32),
        pltpu.SemaphoreType.DMA,
    ],
    compiler_params=pltpu.CompilerParams(needs_layout_passes=False),
)
def _embed_stream(idx_hbm, *rest):
    wts = rest[:NUM_TABLES]
    wtail_hbm = rest[NUM_TABLES]
    out_hbm = rest[NUM_TABLES + 1]
    (wtva, wtvb, wtl, idxv, hmain, htail, stg, zb, spm, sem) = rest[NUM_TABLES + 2:]

    cid = lax.axis_index("c")
    sid = lax.axis_index("s")
    lanes = lax.iota(jnp.int32, L)
    zerov = jnp.zeros((L,), jnp.float32)
    onev = _splat(1)
    zv = _splat(0)
    v4095 = _splat(4095)
    v3 = _splat(3)
    vE = _splat(EMBED)
    v2c = _splat(2)
    v31 = _splat(31)
    vtrash = _splat(TRASH)
    vchunkm1 = _splat(CHUNK - 1)

    for r in range(L):
        for cc in range(8):
            zb[r, pl.ds(cc * L, L)] = zerov

    @pl.when(sid == 0)
    def _():
        pltpu.sync_copy(zb.at[pl.ds(0, 8), :], spm.at[pl.ds(PROWS, 8), :])
        pltpu.sync_copy(zb, spm.at[pl.ds(ZB_ROW, L), :])

    def do_field(f, fl):
        c0 = sid * 2
        c1 = sid * 2 + 1
        base0 = jnp.minimum(c0 * CHUNK, MAXBASE)
        base1 = jnp.minimum(c1 * CHUNK, MAXBASE)
        lo0 = c0 * CHUNK
        hi0 = jnp.minimum(lo0 + CHUNK, HICAP)
        lo1 = c1 * CHUNK
        hi1 = jnp.minimum(lo1 + CHUNK, HICAP)

        vbase0 = _splat(base0)
        vlo0 = _splat(lo0)
        vhi0 = _splat(hi0)
        vlo1 = _splat(lo1)
        vhi1 = _splat(hi1)
        vd01 = _splat((base1 - base0) * 4096)
        vdt = _splat((HICAP - base0) * 4096)
        vgate = _splat(jnp.where(c1 == 31, 1, 0))
        v12 = _splat(12)
        vhicap = _splat(HICAP)
        vfl = _splat(fl * (BATCH // 4))

        f0, f1 = f, FPSC + f
        b0 = pl.multiple_of(base0, 128)
        b1 = pl.multiple_of(base1, 128)

        @pl.when(cid == 0)
        def _():
            pltpu.sync_copy(idx_hbm.at[f0], idxv)
            pltpu.sync_copy(wtail_hbm.at[f0], wtl)

        @pl.when(cid == 1)
        def _():
            pltpu.sync_copy(idx_hbm.at[f1], idxv)
            pltpu.sync_copy(wtail_hbm.at[f1], wtl)

        def load(dst, rlo, bb):
            def issue0():
                return pltpu.async_copy(
                    wts[f0].at[pl.ds(rlo, L), pl.ds(bb, CHUNK)], dst, sem)

            def issue1():
                return pltpu.async_copy(
                    wts[f1].at[pl.ds(rlo, L), pl.ds(bb, CHUNK)], dst, sem)

            @pl.when(cid == 0)
            def _():
                issue0()

            @pl.when(cid == 1)
            def _():
                issue1()

            return pltpu.make_async_copy(
                wts[f0].at[pl.ds(rlo, L), pl.ds(bb, CHUNK)], dst, sem)

        ld_a0 = load(wtva, 0, b0)
        ld_b0 = load(wtvb, L, b0)

        vw0 = _splat(hi0 - lo0)
        vw1 = _splat(hi1 - lo1)

        def scan_body(g, carry):
            off0, off1, offt = carry
            rowv = _splat(g >> 3)
            colv = _splat((g & 7) * L) + lanes
            iv = plsc.load_gather(idxv, [rowv, colv])
            bv = _splat(g * L) + lanes
            d0 = iv - vlo0
            h0 = ((iv - vbase0) << v12) | bv
            m0 = plsc.bitcast(d0, jnp.uint32) < plsc.bitcast(vw0, jnp.uint32)
            m1 = (plsc.bitcast(d0 - vw0, jnp.uint32)
                  < plsc.bitcast(vw1, jnp.uint32))

            cs0 = plsc.cumsum(jnp.where(m0, onev, zv))
            plsc.store_scatter(hmain, [off0 + cs0 - onev], h0, mask=m0)
            off0 = off0 + plsc.all_reduce_population_count(m0)

            cs1 = plsc.cumsum(jnp.where(m1, onev, zv))
            plsc.store_scatter(
                hmain, [off1 - cs1 + onev], h0 - vd01, mask=m1)
            off1 = off1 - plsc.all_reduce_population_count(m1)

            mt = (iv >= vhicap) & (vgate > zv)
            cst = plsc.cumsum(jnp.where(mt, onev, zv))
            plsc.store_scatter(htail, [offt + cst - onev], h0 - vdt, mask=mt)
            offt = offt + plsc.all_reduce_population_count(mt)
            return (off0, off1, offt)

        off0, off1, offt = lax.fori_loop(
            0, BATCH // L, scan_body, (zv, _splat(CAP - 1), zv))
        cnt0 = jnp.max(off0)
        cnt1 = (CAP - 1) - jnp.max(off1)
        cntt = jnp.max(offt)

        def extract_main(hbase, step, cnt, buf, c_off):
            def ex_body(g, _):
                start = hbase + g * L if step > 0 else hbase - g * L - (L - 1)
                hv = hmain[pl.ds(start, L)]
                rv = jnp.minimum((hv >> v12) & v4095, vchunkm1)
                bvv = hv & v4095
                if step > 0:
                    pos = _splat(g * L) + lanes
                else:
                    pos = _splat(g * L + L - 1) - lanes
                valid = pos < _splat(cnt)
                pltpu.sync_copy(spm.at[pl.ds(ZB_ROW, L), :], stg)

                def c_body(c, _2):
                    cv = _splat(c)
                    vals = plsc.load_gather(buf, [cv, rv])
                    dstc = (bvv & v3) * vE + cv + _splat(c_off)
                    plsc.store_scatter(stg, [lanes, dstc], vals, mask=valid)
                    return 0

                lax.fori_loop(0, L, c_body, 0)
                drows = vfl + (bvv >> v2c)
                drows = jnp.where(valid, drows, vtrash)
                pltpu.sync_copy(stg, spm.at[drows], add=True)
                return 0

            lax.fori_loop(0, (cnt + L - 1) >> 4, ex_body, 0)

        def extract_tail(cnt):
            def ex_body(g, _):
                hv = htail[pl.ds(g * L, L)]
                rv = (hv >> v12) & v31
                bvv = hv & v4095
                pos = _splat(g * L) + lanes
                valid = pos < _splat(cnt)
                pltpu.sync_copy(spm.at[pl.ds(ZB_ROW, L), :], stg)

                def c_body(c, _2):
                    cv = _splat(c)
                    fpos = rv * _splat(EMBED) + cv
                    vals = plsc.load_gather(
                        wtl, [fpos >> _splat(7), fpos & _splat(127)])
                    dstc = (bvv & v3) * vE + cv
                    plsc.store_scatter(stg, [lanes, dstc], vals, mask=valid)
                    return 0

                lax.fori_loop(0, EMBED, c_body, 0)
                drows = vfl + (bvv >> v2c)
                drows = jnp.where(valid, drows, vtrash)
                pltpu.sync_copy(stg, spm.at[drows], add=True)
                return 0

            lax.fori_loop(0, (cnt + L - 1) >> 4, ex_body, 0)

        ld_a0.wait()
        extract_main(jnp.int32(0), 1, cnt0, wtva, 0)
        ld_a1 = load(wtva, 0, b1)
        ld_b0.wait()
        extract_main(jnp.int32(0), 1, cnt0, wtvb, L)
        ld_b1 = load(wtvb, L, b1)
        ld_a1.wait()
        extract_main(jnp.int32(CAP - 1), -1, cnt1, wtva, 0)
        ld_b1.wait()
        extract_main(jnp.int32(CAP - 1), -1, cnt1, wtvb, L)
        extract_tail(cntt)

    for f_start, nf in PASSES:
        share = nf * (BATCH // 4) // L
        for r in range(share // L):
            pltpu.sync_copy(zb, spm.at[pl.ds(sid * share + r * L, L), :])
        plsc.subcore_barrier()
        do_field(f_start, 0)
        plsc.subcore_barrier()
        row0 = cid * ROWS_SC + f_start * (BATCH // 4) + sid * share
        pltpu.sync_copy(
            spm.at[pl.ds(sid * share, share), :],
            out_hbm.at[pl.ds(row0, share), :])


def kernel(
    feat_0, feat_1, feat_2, feat_3, feat_4, feat_5, feat_6, feat_7,
    feat_8, feat_9, feat_10, feat_11, feat_12, feat_13, feat_14, feat_15,
    feat_16, feat_17, feat_18, feat_19, feat_20, feat_21, feat_22, feat_23,
    feat_24, feat_25,
    W_0, W_1, W_2, W_3, W_4, W_5, W_6, W_7,
    W_8, W_9, W_10, W_11, W_12, W_13, W_14, W_15,
    W_16, W_17, W_18, W_19, W_20, W_21, W_22, W_23,
    W_24, W_25,
):
    feats = [
        feat_0, feat_1, feat_2, feat_3, feat_4, feat_5, feat_6, feat_7,
        feat_8, feat_9, feat_10, feat_11, feat_12, feat_13, feat_14, feat_15,
        feat_16, feat_17, feat_18, feat_19, feat_20, feat_21, feat_22,
        feat_23, feat_24, feat_25,
    ]
    ws = [
        W_0, W_1, W_2, W_3, W_4, W_5, W_6, W_7,
        W_8, W_9, W_10, W_11, W_12, W_13, W_14, W_15,
        W_16, W_17, W_18, W_19, W_20, W_21, W_22, W_23,
        W_24, W_25,
    ]
    idx3 = jnp.stack(feats).reshape(NUM_TABLES, 32, 128)
    wtail = jnp.stack([w[HICAP:] for w in ws]).reshape(
        NUM_TABLES, TAIL // 4, 4 * EMBED)
    out2 = _embed_stream(idx3, *[w.T for w in ws], wtail)
    return (
        out2.reshape(NUM_TABLES, BATCH // 4, 4, EMBED)
        .transpose(1, 2, 0, 3)
        .reshape(BATCH, OUT_D)
    )


# speedup vs baseline: 1.0417x; 1.0417x over previous
import functools

import jax
import jax.numpy as jnp
from jax import lax
from jax.experimental import pallas as pl
from jax.experimental.pallas import tpu as pltpu
from jax.experimental.pallas import tpu_sc as plsc

NUM_TABLES = 26
VOCAB = 100000
EMBED = 32
BATCH = 4096
OUT_D = NUM_TABLES * EMBED
L = 16

FPSC = NUM_TABLES // 2
CHUNK = 3200
MAXBASE = 96768
HICAP = MAXBASE + CHUNK
TAIL = VOCAB - HICAP
ROWS_SC = FPSC * (BATCH // 4)
PASSES = tuple((i, 1) for i in range(FPSC))
PROWS = BATCH // 4
TRASH = PROWS
ZB_ROW = PROWS + 8
CAP = BATCH

_info = plsc.get_sparse_core_info()
_NS = _info.num_subcores

_mesh = plsc.VectorSubcoreMesh(core_axis_name="c", subcore_axis_name="s")


def _splat(x):
    return jnp.full((L,), x, jnp.int32)


@functools.partial(
    pl.kernel,
    mesh=_mesh,
    out_type=jax.ShapeDtypeStruct((2 * ROWS_SC, 4 * EMBED), jnp.float32),
    scratch_types=[
        pltpu.VMEM((32, CHUNK), jnp.float32),
        pltpu.VMEM((TAIL // 4, 4 * EMBED), jnp.float32),
        pltpu.VMEM((32, 128), jnp.int32),
        pltpu.VMEM((CAP,), jnp.int32),
        pltpu.VMEM((CAP,), jnp.int32),
        pltpu.VMEM((L, 4 * EMBED), jnp.float32),
        pltpu.VMEM((L, 4 * EMBED), jnp.float32),
        pltpu.VMEM_SHARED((PROWS + 8 + L, 4 * EMBED), jnp.float32),
        pltpu.SemaphoreType.DMA,
    ],
    compiler_params=pltpu.CompilerParams(needs_layout_passes=False),
)
def _embed_stream(idx_hbm, *rest):
    wts = rest[:NUM_TABLES]
    wtail_hbm = rest[NUM_TABLES]
    out_hbm = rest[NUM_TABLES + 1]
    (wtv, wtl, idxv, hmain, htail, stg, zb, spm, sem) = rest[NUM_TABLES + 2:]

    cid = lax.axis_index("c")
    sid = lax.axis_index("s")
    lanes = lax.iota(jnp.int32, L)
    zerov = jnp.zeros((L,), jnp.float32)
    onev = _splat(1)
    zv = _splat(0)

    for r in range(L):
        for cc in range(8):
            zb[r, pl.ds(cc * L, L)] = zerov

    @pl.when(sid == 0)
    def _():
        pltpu.sync_copy(zb.at[pl.ds(0, 8), :], spm.at[pl.ds(PROWS, 8), :])
        pltpu.sync_copy(zb, spm.at[pl.ds(ZB_ROW, L), :])

    def do_field(f, fl):
        c0 = sid * 2
        c1 = sid * 2 + 1
        base0 = jnp.minimum(c0 * CHUNK, MAXBASE)
        base1 = jnp.minimum(c1 * CHUNK, MAXBASE)
        lo0 = c0 * CHUNK
        hi0 = jnp.minimum(lo0 + CHUNK, HICAP)
        lo1 = c1 * CHUNK
        hi1 = jnp.minimum(lo1 + CHUNK, HICAP)

        vbase0 = _splat(base0)
        vlo0 = _splat(lo0)
        vhi0 = _splat(hi0)
        vlo1 = _splat(lo1)
        vhi1 = _splat(hi1)
        vd01 = _splat((base1 - base0) * 4096)
        vdt = _splat((HICAP - base0) * 4096)
        vgate = _splat(jnp.where(c1 == 31, 1, 0))
        v12 = _splat(12)
        vhicap = _splat(HICAP)

        pltpu.sync_copy(idx_hbm.at[f], idxv)
        wload = pltpu.async_copy(
            wts[f].at[:, pl.ds(pl.multiple_of(base0, 128), CHUNK)], wtv, sem)

        vw0 = _splat(hi0 - lo0)
        vw1 = _splat(hi1 - lo1)

        def scan_body(g, carry):
            off0, off1, offt = carry
            rowv = _splat(g >> 3)
            colv = _splat((g & 7) * L) + lanes
            iv = plsc.load_gather(idxv, [rowv, colv])
            bv = _splat(g * L) + lanes
            d0 = iv - vlo0
            h0 = ((iv - vbase0) << v12) | bv
            m0 = plsc.bitcast(d0, jnp.uint32) < plsc.bitcast(vw0, jnp.uint32)
            m1 = (plsc.bitcast(d0 - vw0, jnp.uint32)
                  < plsc.bitcast(vw1, jnp.uint32))

            cs0 = plsc.cumsum(jnp.where(m0, onev, zv))
            plsc.store_scatter(hmain, [off0 + cs0 - onev], h0, mask=m0)
            off0 = off0 + plsc.all_reduce_population_count(m0)

            cs1 = plsc.cumsum(jnp.where(m1, onev, zv))
            plsc.store_scatter(
                hmain, [off1 - cs1 + onev], h0 - vd01, mask=m1)
            off1 = off1 - plsc.all_reduce_population_count(m1)

            mt = (iv >= vhicap) & (vgate > zv)
            cst = plsc.cumsum(jnp.where(mt, onev, zv))
            plsc.store_scatter(htail, [offt + cst - onev], h0 - vdt, mask=mt)
            offt = offt + plsc.all_reduce_population_count(mt)
            return (off0, off1, offt)

        off0, off1, offt = lax.fori_loop(
            0, BATCH // L, scan_body, (zv, _splat(CAP - 1), zv))
        cnt0 = jnp.max(off0)
        cnt1 = (CAP - 1) - jnp.max(off1)
        cntt = jnp.max(offt)

        def extract_main(hbase, step, cnt):
            def ex_body(g, _):
                start = hbase + g * L if step > 0 else hbase - g * L - (L - 1)
                hv = hmain[pl.ds(start, L)]
                rv = jnp.minimum((hv >> v12) & _splat(4095), _splat(CHUNK - 1))
                bvv = hv & _splat(4095)
                if step > 0:
                    pos = _splat(g * L) + lanes
                else:
                    pos = _splat(g * L + L - 1) - lanes
                valid = pos < _splat(cnt)
                pltpu.sync_copy(spm.at[pl.ds(ZB_ROW, L), :], stg)

                def c_body(c, _2):
                    cv = _splat(c)
                    vals = plsc.load_gather(wtv, [cv, rv])
                    dstc = (bvv & _splat(3)) * _splat(EMBED) + cv
                    plsc.store_scatter(stg, [lanes, dstc], vals, mask=valid)
                    return 0

                lax.fori_loop(0, EMBED, c_body, 0)
                drows = _splat(fl * (BATCH // 4)) + (bvv >> _splat(2))
                drows = jnp.where(valid, drows, _splat(TRASH))
                pltpu.sync_copy(stg, spm.at[drows], add=True)
                return 0

            lax.fori_loop(0, (cnt + L - 1) >> 4, ex_body, 0)

        def extract_tail(cnt):
            def ex_body(g, _):
                hv = htail[pl.ds(g * L, L)]
                rv = (hv >> v12) & _splat(31)
                bvv = hv & _splat(4095)
                pos = _splat(g * L) + lanes
                valid = pos < _splat(cnt)
                pltpu.sync_copy(spm.at[pl.ds(ZB_ROW, L), :], stg)

                def c_body(c, _2):
                    cv = _splat(c)
                    fpos = rv * _splat(EMBED) + cv
                    vals = plsc.load_gather(
                        wtl, [fpos >> _splat(7), fpos & _splat(127)])
                    dstc = (bvv & _splat(3)) * _splat(EMBED) + cv
                    plsc.store_scatter(stg, [lanes, dstc], vals, mask=valid)
                    return 0

                lax.fori_loop(0, EMBED, c_body, 0)
                drows = _splat(fl * (BATCH // 4)) + (bvv >> _splat(2))
                drows = jnp.where(valid, drows, _splat(TRASH))
                pltpu.sync_copy(stg, spm.at[drows], add=True)
                return 0

            lax.fori_loop(0, (cnt + L - 1) >> 4, ex_body, 0)

        wload.wait()
        extract_main(jnp.int32(0), 1, cnt0)
        pltpu.sync_copy(
            wts[f].at[:, pl.ds(pl.multiple_of(base1, 128), CHUNK)], wtv)
        extract_main(jnp.int32(CAP - 1), -1, cnt1)
        pltpu.sync_copy(wtail_hbm.at[f], wtl)
        extract_tail(cntt)

    for f_start, nf in PASSES:
        share = nf * (BATCH // 4) // L
        for r in range(share // L):
            pltpu.sync_copy(zb, spm.at[pl.ds(sid * share + r * L, L), :])
        plsc.subcore_barrier()

        @pl.when(cid == 0)
        def _():
            for fl in range(f_start, f_start + nf):
                do_field(fl, fl - f_start)

        @pl.when(cid == 1)
        def _():
            for fl in range(f_start, f_start + nf):
                do_field(FPSC + fl, fl - f_start)

        plsc.subcore_barrier()
        row0 = cid * ROWS_SC + f_start * (BATCH // 4) + sid * share
        pltpu.sync_copy(
            spm.at[pl.ds(sid * share, share), :],
            out_hbm.at[pl.ds(row0, share), :])


def kernel(
    feat_0, feat_1, feat_2, feat_3, feat_4, feat_5, feat_6, feat_7,
    feat_8, feat_9, feat_10, feat_11, feat_12, feat_13, feat_14, feat_15,
    feat_16, feat_17, feat_18, feat_19, feat_20, feat_21, feat_22, feat_23,
    feat_24, feat_25,
    W_0, W_1, W_2, W_3, W_4, W_5, W_6, W_7,
    W_8, W_9, W_10, W_11, W_12, W_13, W_14, W_15,
    W_16, W_17, W_18, W_19, W_20, W_21, W_22, W_23,
    W_24, W_25,
):
    feats = [
        feat_0, feat_1, feat_2, feat_3, feat_4, feat_5, feat_6, feat_7,
        feat_8, feat_9, feat_10, feat_11, feat_12, feat_13, feat_14, feat_15,
        feat_16, feat_17, feat_18, feat_19, feat_20, feat_21, feat_22,
        feat_23, feat_24, feat_25,
    ]
    ws = [
        W_0, W_1, W_2, W_3, W_4, W_5, W_6, W_7,
        W_8, W_9, W_10, W_11, W_12, W_13, W_14, W_15,
        W_16, W_17, W_18, W_19, W_20, W_21, W_22, W_23,
        W_24, W_25,
    ]
    idx3 = jnp.stack(feats).reshape(NUM_TABLES, 32, 128)
    wtail = jnp.stack([w[HICAP:] for w in ws]).reshape(
        NUM_TABLES, TAIL // 4, 4 * EMBED)
    out2 = _embed_stream(idx3, *[w.T for w in ws], wtail)
    return (
        out2.reshape(NUM_TABLES, BATCH // 4, 4, EMBED)
        .transpose(1, 2, 0, 3)
        .reshape(BATCH, OUT_D)
    )
